# Initial kernel scaffold; baseline (speedup 1.0000x reference)
import jax
import jax.numpy as jnp
from jax.experimental import pallas as pl


def kernel(session_ids, history_ids, target_ids, session_table, hist_table, att_W1, att_b1, att_W2, att_b2, mlp_W1, mlp_b1, alpha1, mlp_W2, mlp_b2, alpha2, mlp_W3, mlp_b3):
    raise NotImplementedError("write your pallas kernel here")


# speedup vs baseline: 1.4880x; 1.4880x over previous
import functools

import jax
import jax.numpy as jnp
from jax import lax
from jax.experimental import pallas as pl
from jax.experimental.pallas import tpu as pltpu
from jax.experimental.pallas import tpu_sc as plsc

B = 4096
D = 32
N_SESS = 24
N_HIST = 2
L = 50
LP = 56
ATT_HID = 36
H1, H2 = 256, 128

NC, NS = 2, 16
NW = NC * NS

TB = 256
NB = B // TB


def _sc_gather_all(sess_idx, hist_idx, tgt_idx, sess_tab, hist_tab):
    n_s, n_h, n_t = sess_idx.shape[0], hist_idx.shape[0], tgt_idx.shape[0]
    sp, hp, tp = n_s // NW, n_h // NW, n_t // NW
    CH = 1024
    f32 = jnp.float32
    mesh = plsc.VectorSubcoreMesh(core_axis_name="c", subcore_axis_name="s")

    @functools.partial(
        pl.kernel, mesh=mesh,
        out_type=(jax.ShapeDtypeStruct((n_s, D), f32),
                  jax.ShapeDtypeStruct((n_h, D), f32),
                  jax.ShapeDtypeStruct((n_t, D), f32)),
        scratch_types=[pltpu.VMEM((hp,), jnp.int32),
                       pltpu.VMEM((CH, D), f32),
                       pltpu.VMEM((CH, D), f32),
                       pltpu.SemaphoreType.DMA,
                       pltpu.SemaphoreType.DMA],
        compiler_params=pltpu.CompilerParams(use_tc_tiling_on_sc=False),
    )
    def k(sidx, hidx, tidx, stab, htab, s_out, h_out, t_out,
          idx_v, rows0, rows1, sem0, sem1):
        wid = lax.axis_index("s") * NC + lax.axis_index("c")
        bufs = (rows0, rows1)
        sems = (sem0, sem1)

        def gather_stream(idx_hbm, tab, out, per, chunk):
            base = wid * per
            n = per // chunk
            pltpu.sync_copy(idx_hbm.at[pl.ds(base, per)],
                            idx_v.at[pl.ds(0, per)])
            cps = [None, None]
            cps[0] = pltpu.make_async_copy(
                tab.at[idx_v.at[pl.ds(0, chunk)]],
                bufs[0].at[pl.ds(0, chunk)], sems[0])
            cps[0].start()
            for j in range(n):
                cur = j % 2
                if j + 1 < n:
                    nxt = (j + 1) % 2
                    cps[nxt] = pltpu.make_async_copy(
                        tab.at[idx_v.at[pl.ds((j + 1) * chunk, chunk)]],
                        bufs[nxt].at[pl.ds(0, chunk)], sems[nxt])
                    cps[nxt].start()
                cps[cur].wait()
                pltpu.sync_copy(bufs[cur].at[pl.ds(0, chunk)],
                                out.at[pl.ds(base + j * chunk, chunk)])

        gather_stream(hidx, htab, h_out, hp, CH)
        gather_stream(sidx, stab, s_out, sp, CH)
        gather_stream(tidx, htab, t_out, tp, tp)

    return k(sess_idx, hist_idx, tgt_idx, sess_tab, hist_tab)


def _tc_dense(emb_h4, emb_s2, emb_t2, Wh, Wt, Wm, ab1, w2v,
              W1p0, W1p1, W1s, W1t, b1r, al1, W2, b2r, al2, W3, b3r):
    f32 = jnp.float32

    def body(h_ref, s_ref, t_ref, Wh_ref, Wt_ref, Wm_ref, ab1_ref, w2_ref,
             W1p0_ref, W1p1_ref, W1s_ref, W1t_ref, b1_ref, al1_ref,
             W2_ref, b2_ref, al2_ref, W3_ref, b3_ref, out_ref, x1_scr):
        i = pl.program_id(0)
        t2 = t_ref[...]
        x1 = jnp.dot(s_ref[...], W1s_ref[...], preferred_element_type=f32)
        x1 += jnp.dot(t2, W1t_ref[...], preferred_element_type=f32)
        for br in range(N_HIST):
            h3 = h_ref[:, br]
            t = t2[:, br * D:(br + 1) * D]
            tb3 = jnp.broadcast_to(t[:, None, :], (TB, LP, D))
            hf = h3.reshape(TB * LP, D)
            mf = (h3 * tb3).reshape(TB * LP, D)
            a = jnp.dot(hf, Wh_ref[br], preferred_element_type=f32)
            a += jnp.dot(mf, Wm_ref[br], preferred_element_type=f32)
            tw = jnp.dot(t, Wt_ref[br], preferred_element_type=f32)
            tw = tw + ab1_ref[br]
            a = a.reshape(TB, LP, ATT_HID) + tw[:, None, :]
            a = jnp.maximum(a, 0.0)
            s = jnp.sum(a * w2_ref[br][None], axis=-1)
            mask = lax.broadcasted_iota(jnp.int32, (TB, LP), 1) < L
            s = jnp.where(mask, s, -1e30)
            s = s - jnp.max(s, axis=-1, keepdims=True)
            es = jnp.exp(s)
            w = es / jnp.sum(es, axis=-1, keepdims=True)
            pooled = jnp.sum(w[:, :, None] * h3, axis=1)
            wp = W1p0_ref if br == 0 else W1p1_ref
            x1 += jnp.dot(pooled, wp[...], preferred_element_type=f32)
        x1 += b1_ref[...]
        x1_scr[pl.ds(i * TB, TB), :] = x1

        @pl.when(i == NB - 1)
        def _():
            eps = 1e-8
            x = x1_scr[...]
            m = jnp.sum(x, axis=0, keepdims=True) * (1.0 / B)
            v = jnp.sum((x - m) ** 2, axis=0, keepdims=True) * (1.0 / B)
            ps = jax.nn.sigmoid((x - m) * lax.rsqrt(v + eps))
            x = ps * x + (1.0 - ps) * al1_ref[...] * x
            x = jnp.dot(x, W2_ref[...], preferred_element_type=f32) + b2_ref[...]
            m2 = jnp.sum(x, axis=0, keepdims=True) * (1.0 / B)
            v2 = jnp.sum((x - m2) ** 2, axis=0, keepdims=True) * (1.0 / B)
            ps2 = jax.nn.sigmoid((x - m2) * lax.rsqrt(v2 + eps))
            x = ps2 * x + (1.0 - ps2) * al2_ref[...] * x
            y = jnp.dot(x, W3_ref[...], preferred_element_type=f32) + b3_ref[...]
            out_ref[...] = jax.nn.sigmoid(y)

    const3 = lambda shape: pl.BlockSpec(shape, lambda i: (0,) * len(shape))
    return pl.pallas_call(
        body,
        grid=(NB,),
        in_specs=[
            pl.BlockSpec((TB, N_HIST, LP, D), lambda i: (i, 0, 0, 0)),
            pl.BlockSpec((TB, N_SESS * D), lambda i: (i, 0)),
            pl.BlockSpec((TB, N_HIST * D), lambda i: (i, 0)),
            const3((N_HIST, D, ATT_HID)),
            const3((N_HIST, D, ATT_HID)),
            const3((N_HIST, D, ATT_HID)),
            const3((N_HIST, 1, ATT_HID)),
            const3((N_HIST, 1, ATT_HID)),
            const3((D, H1)),
            const3((D, H1)),
            const3((N_SESS * D, H1)),
            const3((N_HIST * D, H1)),
            const3((1, H1)),
            const3((1, H1)),
            const3((H1, H2)),
            const3((1, H2)),
            const3((1, H2)),
            const3((H2, 1)),
            const3((1, 1)),
        ],
        out_specs=pl.BlockSpec((B, 1), lambda i: (0, 0)),
        out_shape=jax.ShapeDtypeStruct((B, 1), f32),
        scratch_shapes=[pltpu.VMEM((B, H1), f32)],
        compiler_params=pltpu.CompilerParams(
            dimension_semantics=("arbitrary",)),
    )(emb_h4, emb_s2, emb_t2, Wh, Wt, Wm, ab1, w2v,
      W1p0, W1p1, W1s, W1t, b1r, al1, W2, b2r, al2, W3, b3r)


def kernel(session_ids, history_ids, target_ids, session_table, hist_table,
           att_W1, att_b1, att_W2, att_b2,
           mlp_W1, mlp_b1, alpha1, mlp_W2, mlp_b2, alpha2, mlp_W3, mlp_b3):
    sess_idx = session_ids.reshape(-1).astype(jnp.int32)
    hist_pad = jnp.pad(history_ids.astype(jnp.int32),
                       ((0, 0), (0, 0), (0, LP - L))).reshape(-1)
    tgt_idx = target_ids.reshape(-1).astype(jnp.int32)

    emb_s, emb_h, emb_t = _sc_gather_all(
        sess_idx, hist_pad, tgt_idx, session_table, hist_table)
    emb_h4 = emb_h.reshape(B, N_HIST, LP, D)
    emb_s2 = emb_s.reshape(B, N_SESS * D)
    emb_t2 = emb_t.reshape(B, N_HIST * D)

    Wh = att_W1[:, 0:D] + att_W1[:, 2 * D:3 * D]
    Wt = att_W1[:, D:2 * D] - att_W1[:, 2 * D:3 * D]
    Wm = att_W1[:, 3 * D:4 * D]
    ab1 = att_b1.reshape(N_HIST, 1, ATT_HID)
    w2v = att_W2.reshape(N_HIST, 1, ATT_HID)

    W1p0 = mlp_W1[0:D]
    W1p1 = mlp_W1[D:2 * D]
    W1s = mlp_W1[2 * D:2 * D + N_SESS * D]
    W1t = mlp_W1[2 * D + N_SESS * D:]
    b1r = mlp_b1.reshape(1, H1)
    al1 = alpha1.reshape(1, H1)
    b2r = mlp_b2.reshape(1, H2)
    al2 = alpha2.reshape(1, H2)
    b3r = mlp_b3.reshape(1, 1)

    out = _tc_dense(emb_h4, emb_s2, emb_t2, Wh, Wt, Wm, ab1, w2v,
                    W1p0, W1p1, W1s, W1t, b1r, al1,
                    mlp_W2, b2r, al2, mlp_W3, b3r)
    return out.reshape(B)
